# baseline (speedup 1.0000x reference)
import jax
import jax.numpy as jnp
from jax.experimental import pallas as pl

N = 10000
R = 8


def _rgcn_conv(x, edge_index, edge_type, weight, root, bias):
    n = x.shape[0]
    r, nb, bin_, bout = weight.shape
    out_ch = nb * bout
    src = edge_index[0]
    dst = edge_index[1]
    h = x.reshape(n, nb, bin_)
    H_all = jnp.einsum('nbc,rbcd->rnbd', h, weight).reshape(r, n, out_ch)
    msg = H_all[edge_type, src]
    seg = dst * r + edge_type
    sums = jax.ops.segment_sum(msg, seg, num_segments=n * r)
    cnts = jax.ops.segment_sum(jnp.ones((msg.shape[0],), dtype=msg.dtype), seg, num_segments=n * r)
    mean = sums / jnp.maximum(cnts, 1.0)[:, None]
    agg = mean.reshape(n, r, out_ch).sum(axis=1)
    return agg + x @ root


def _add_kernel(a_ref, b_ref, o_ref):
    o_ref[...] = a_ref[...] + b_ref[...]


def kernel(x, edge_index, edge_type, W1, b1, w1, root1, bias1, w2, root2, bias2):
    h = x @ W1 + b1
    h = jax.nn.relu(_rgcn_conv(h, edge_index, edge_type, w1, root1, bias1) + bias1)
    out = _rgcn_conv(h, edge_index, edge_type, w2, root2, bias2)
    out = pl.pallas_call(
        _add_kernel,
        out_shape=jax.ShapeDtypeStruct(out.shape, out.dtype),
    )(out, jnp.broadcast_to(bias2, out.shape))
    return out


# speedup vs baseline: 3.3728x; 3.3728x over previous
import functools

import jax
import jax.numpy as jnp
from jax import lax
from jax.experimental import pallas as pl
from jax.experimental.pallas import tpu as pltpu
from jax.experimental.pallas import tpu_sc as plsc

N = 10000
E = 160000
IN = 256
HID = 500
OUT = 250
R = 8
NB = 5
P = 256
HP = 128
NC = 2
NS = 16
CH = 128
NCH = E // CH
NRP = 81920
SEG_T = NRP // NS
RB = 1000

_f32 = jnp.float32
_i32 = jnp.int32


def _zvec():
    return jnp.zeros((16,), _f32)


def _dyn_gather(v, idx):
    dnums = lax.GatherDimensionNumbers(
        offset_dims=(), collapsed_slice_dims=(0,), start_index_map=(0,))
    return lax.gather(v, idx[:, None], dnums, (1,),
                      mode=lax.GatherScatterMode.PROMISE_IN_BOUNDS)


def _cnt_body(src_hbm, dst_hbm, rel_hbm,
              invc_hbm, idx2_hbm,
              src_v, dst_v, rel_v, seg_v, ones_v, f_v, cbuf_v, cnt_sh):
    c = lax.axis_index("c")
    s = lax.axis_index("s")

    for j in range(CH // 16):
        ones_v[pl.ds(j * 16, 16)] = jnp.ones((16,), _f32)

    @pl.loop(0, SEG_T // 16)
    def _zero(i):
        cbuf_v[pl.ds(pl.multiple_of(i * 16, 16), 16)] = _zvec()

    pltpu.sync_copy(cbuf_v, cnt_sh.at[pl.ds(s * SEG_T, SEG_T)])
    plsc.subcore_barrier()

    nck = jnp.where(s < (NCH % NS), NCH // NS + 1, NCH // NS)

    @pl.loop(0, nck)
    def _count(i):
        base = pl.multiple_of((s + NS * i) * CH, CH)
        pltpu.sync_copy(dst_hbm.at[pl.ds(base, CH)], dst_v)
        pltpu.sync_copy(rel_hbm.at[pl.ds(base, CH)], rel_v)
        for j in range(CH // 16):
            sl = pl.ds(j * 16, 16)
            seg_v[sl] = dst_v[sl] * R + rel_v[sl]
        pltpu.sync_copy(ones_v, cnt_sh.at[seg_v], add=True)

    plsc.subcore_barrier()

    pltpu.sync_copy(cnt_sh.at[pl.ds(s * SEG_T, SEG_T)], cbuf_v)

    @pl.loop(0, SEG_T // 16)
    def _recip(i):
        sl = pl.ds(pl.multiple_of(i * 16, 16), 16)
        cbuf_v[sl] = 1.0 / jnp.maximum(cbuf_v[sl], 1.0)

    pltpu.sync_copy(cbuf_v, cnt_sh.at[pl.ds(s * SEG_T, SEG_T)])
    plsc.subcore_barrier()

    w = s * NC + c
    NW = NC * NS
    nck2 = jnp.where(w < (NCH % NW), NCH // NW + 1, NCH // NW)

    @pl.loop(0, nck2)
    def _edges(i):
        base = pl.multiple_of((w + NW * i) * CH, CH)
        pltpu.sync_copy(src_hbm.at[pl.ds(base, CH)], src_v)
        pltpu.sync_copy(dst_hbm.at[pl.ds(base, CH)], dst_v)
        pltpu.sync_copy(rel_hbm.at[pl.ds(base, CH)], rel_v)
        for j in range(CH // 16):
            sl = pl.ds(j * 16, 16)
            seg_v[sl] = dst_v[sl] * R + rel_v[sl]
            g2 = (rel_v[sl] * N + src_v[sl]) * 2
            dst_v[sl] = g2
            src_v[sl] = g2 + 1
        pltpu.sync_copy(cnt_sh.at[seg_v], f_v)
        pltpu.sync_copy(f_v, invc_hbm.at[pl.ds(base, CH)])
        pltpu.sync_copy(dst_v, idx2_hbm.at[0, pl.ds(base, CH)])
        pltpu.sync_copy(src_v, idx2_hbm.at[1, pl.ds(base, CH)])


_cnt_call = pl.kernel(
    _cnt_body,
    out_type=(jax.ShapeDtypeStruct((E,), _f32),
              jax.ShapeDtypeStruct((2, E), _i32)),
    mesh=plsc.VectorSubcoreMesh(core_axis_name="c", subcore_axis_name="s"),
    scratch_types=[
        pltpu.VMEM((CH,), _i32),
        pltpu.VMEM((CH,), _i32),
        pltpu.VMEM((CH,), _i32),
        pltpu.VMEM((CH,), _i32),
        pltpu.VMEM((CH,), _f32),
        pltpu.VMEM((CH,), _f32),
        pltpu.VMEM((SEG_T,), _f32),
        pltpu.VMEM_SHARED((NRP,), _f32),
    ],
)


_RW = 80
_NRW = N // _RW


def _agg_body(tab_hbm, idx2_hbm, invc_hbm, dst_hbm,
              agg_hbm,
              idx_v, dst_v, invc_v, rows_v, sem, acc_sh):
    c = lax.axis_index("c")
    s = lax.axis_index("s")

    @pl.loop(0, CH)
    def _zr(i):
        for j in range(HP // 16):
            rows_v[i, pl.ds(j * 16, 16)] = _zvec()

    nrw = jnp.where(s < (_NRW % NS), _NRW // NS + 1, _NRW // NS)

    @pl.loop(0, nrw)
    def _za(t):
        r0 = pl.multiple_of((s + NS * t) * _RW, _RW)
        pltpu.sync_copy(rows_v.at[pl.ds(0, _RW)], acc_sh.at[pl.ds(r0, _RW)])

    plsc.subcore_barrier()

    nck = jnp.where(s < (NCH % NS), NCH // NS + 1, NCH // NS)

    @pl.loop(0, nck)
    def _main(i):
        base = pl.multiple_of((s + NS * i) * CH, CH)
        pltpu.sync_copy(idx2_hbm.at[c, pl.ds(base, CH)], idx_v)
        pltpu.sync_copy(invc_hbm.at[pl.ds(base, CH)], invc_v)
        pltpu.sync_copy(dst_hbm.at[pl.ds(base, CH)], dst_v)
        pltpu.async_copy(tab_hbm.at[idx_v], rows_v, sem).wait()

        @pl.loop(0, CH // 16)
        def _scale(t):
            iv = invc_v[pl.ds(pl.multiple_of(t * 16, 16), 16)]
            for m in range(16):
                bc = _dyn_gather(iv, jnp.full((16,), m, _i32))
                k = t * 16 + m
                for j in range(HP // 16):
                    sl = pl.ds(j * 16, 16)
                    rows_v[k, sl] = rows_v[k, sl] * bc

        pltpu.sync_copy(rows_v, acc_sh.at[dst_v], add=True)

    plsc.subcore_barrier()

    @pl.loop(0, nrw)
    def _wo(t):
        r0 = pl.multiple_of((s + NS * t) * _RW, _RW)
        pltpu.sync_copy(acc_sh.at[pl.ds(r0, _RW)], rows_v.at[pl.ds(0, _RW)])
        pltpu.sync_copy(rows_v.at[pl.ds(0, _RW)], agg_hbm.at[c, pl.ds(r0, _RW)])


_agg_call = pl.kernel(
    _agg_body,
    out_type=jax.ShapeDtypeStruct((NC, N, HP), _f32),
    mesh=plsc.VectorSubcoreMesh(core_axis_name="c", subcore_axis_name="s"),
    scratch_types=[
        pltpu.VMEM((CH,), _i32),
        pltpu.VMEM((CH,), _i32),
        pltpu.VMEM((CH,), _f32),
        pltpu.VMEM((CH, HP), _f32),
        pltpu.SemaphoreType.DMA,
        pltpu.VMEM_SHARED((N, HP), _f32),
    ],
)


def _ka_body(x_ref, w1_ref, b1_ref, wbd_ref, rootp_ref, hr_ref, ht_ref):
    hb = jnp.dot(x_ref[...], w1_ref[...],
                 preferred_element_type=_f32) + b1_ref[...]
    hr_ref[...] = jnp.dot(hb, rootp_ref[...], preferred_element_type=_f32)
    for r in range(R):
        ht_ref[r, ...] = jnp.dot(hb, wbd_ref[r, ...],
                                 preferred_element_type=_f32)


_ka_call = pl.pallas_call(
    _ka_body,
    grid=(N // RB,),
    in_specs=[
        pl.BlockSpec((RB, IN), lambda i: (i, 0)),
        pl.BlockSpec((IN, HID), lambda i: (0, 0)),
        pl.BlockSpec((1, HID), lambda i: (0, 0)),
        pl.BlockSpec((R, HID, P), lambda i: (0, 0, 0)),
        pl.BlockSpec((HID, P), lambda i: (0, 0)),
    ],
    out_specs=[
        pl.BlockSpec((RB, P), lambda i: (i, 0)),
        pl.BlockSpec((R, RB, P), lambda i: (0, i, 0)),
    ],
    out_shape=[
        jax.ShapeDtypeStruct((N, P), _f32),
        jax.ShapeDtypeStruct((R, N, P), _f32),
    ],
)


def _kb_body(agg_ref, hr1_ref, b1p_ref, wbd_ref, rootp_ref, hr2_ref, ht_ref):
    a = jnp.concatenate([agg_ref[0], agg_ref[1]], axis=1)
    h1 = jnp.maximum(a + hr1_ref[...] + b1p_ref[...], 0.0)
    hr2_ref[...] = jnp.dot(h1, rootp_ref[...], preferred_element_type=_f32)
    for r in range(R):
        ht_ref[r, ...] = jnp.dot(h1, wbd_ref[r, ...],
                                 preferred_element_type=_f32)


_kb_call = pl.pallas_call(
    _kb_body,
    grid=(N // RB,),
    in_specs=[
        pl.BlockSpec((NC, RB, HP), lambda i: (0, i, 0)),
        pl.BlockSpec((RB, P), lambda i: (i, 0)),
        pl.BlockSpec((1, P), lambda i: (0, 0)),
        pl.BlockSpec((R, P, P), lambda i: (0, 0, 0)),
        pl.BlockSpec((P, P), lambda i: (0, 0)),
    ],
    out_specs=[
        pl.BlockSpec((RB, P), lambda i: (i, 0)),
        pl.BlockSpec((R, RB, P), lambda i: (0, i, 0)),
    ],
    out_shape=[
        jax.ShapeDtypeStruct((N, P), _f32),
        jax.ShapeDtypeStruct((R, N, P), _f32),
    ],
)


def _kc_body(agg_ref, hr2_ref, b2p_ref, out_ref):
    a = jnp.concatenate([agg_ref[0], agg_ref[1]], axis=1)
    out_ref[...] = (a + hr2_ref[...] + b2p_ref[...])[:, :OUT]


_kc_call = pl.pallas_call(
    _kc_body,
    grid=(N // RB,),
    in_specs=[
        pl.BlockSpec((NC, RB, HP), lambda i: (0, i, 0)),
        pl.BlockSpec((RB, P), lambda i: (i, 0)),
        pl.BlockSpec((1, P), lambda i: (0, 0)),
    ],
    out_specs=pl.BlockSpec((RB, OUT), lambda i: (i, 0)),
    out_shape=jax.ShapeDtypeStruct((N, OUT), _f32),
)


def kernel(x, edge_index, edge_type, W1, b1, w1, root1, bias1, w2, root2, bias2):
    src = edge_index[0]
    dst = edge_index[1]
    rel = edge_type

    bi1, bo = HID // NB, OUT // NB
    Wbd1 = jnp.zeros((R, HID, P), _f32)
    for b in range(NB):
        Wbd1 = Wbd1.at[:, b * bi1:(b + 1) * bi1, b * bo:(b + 1) * bo].set(w1[:, b])
    bi2 = OUT // NB
    Wbd2 = jnp.zeros((R, P, P), _f32)
    for b in range(NB):
        Wbd2 = Wbd2.at[:, b * bi2:(b + 1) * bi2, b * bo:(b + 1) * bo].set(w2[:, b])
    root1p = jnp.pad(root1, ((0, 0), (0, P - OUT)))
    root2p = jnp.pad(root2, ((0, P - OUT), (0, P - OUT)))
    b1p = jnp.pad(bias1, (0, P - OUT)).reshape(1, P)
    b2p = jnp.pad(bias2, (0, P - OUT)).reshape(1, P)
    b1r = b1.reshape(1, HID)

    invc, idx2 = _cnt_call(src, dst, rel)
    hr1, H1 = _ka_call(x, W1, b1r, Wbd1, root1p)
    agg1 = _agg_call(H1.reshape(2 * R * N, HP), idx2, invc, dst)
    hr2, H2 = _kb_call(agg1, hr1, b1p, Wbd2, root2p)
    agg2 = _agg_call(H2.reshape(2 * R * N, HP), idx2, invc, dst)
    return _kc_call(agg2, hr2, b2p)
